# Initial kernel scaffold; baseline (speedup 1.0000x reference)
import jax
import jax.numpy as jnp
from jax.experimental import pallas as pl


def kernel(x, causal_mask, Wq, bq, Wk, bk, Wv, bv, Wo, bo):
    raise NotImplementedError("write your pallas kernel here")


# speedup vs baseline: 9.0912x; 9.0912x over previous
import math

import jax
import jax.numpy as jnp
from jax.experimental import pallas as pl
from jax.experimental.pallas import tpu as pltpu

_DIM = 2048
_H = 16
_DH = 128
_S = 2048
_KEEP = max(1, int(_S * (1.0 - 0.6)))
_BLK = 256
_NB = _S // _BLK
_SCALE = 1.0 / math.sqrt(_DH)
_NEG = -1e9


def _attn_kernel(x_ref, wq_ref, bq_ref, wk_ref, bk_ref, wv_ref, bv_ref,
                 o_ref, k_s, v_s):
    i = pl.program_id(1)

    @pl.when(i == 0)
    def _():
        xh = x_ref[...]
        k_s[...] = jnp.dot(xh, wk_ref[0], preferred_element_type=jnp.float32) + bk_ref[0]
        v_s[...] = jnp.dot(xh, wv_ref[0], preferred_element_type=jnp.float32) + bv_ref[0]

    xq = x_ref[pl.ds(i * _BLK, _BLK), :]
    q = jnp.dot(xq, wq_ref[0], preferred_element_type=jnp.float32) + bq_ref[0]
    scores = jax.lax.dot_general(
        q, k_s[...], (((1,), (1,)), ((), ())),
        preferred_element_type=jnp.float32) * _SCALE

    rows = i * _BLK + jax.lax.broadcasted_iota(jnp.int32, (_BLK, _S), 0)
    cols = jax.lax.broadcasted_iota(jnp.int32, (_BLK, _S), 1)
    scores = jnp.where(cols <= rows, scores, _NEG)

    u = jax.lax.bitcast_convert_type(scores, jnp.uint32)
    ukey = jnp.where(u >= jnp.uint32(0x80000000), ~u, u | jnp.uint32(0x80000000))
    prefix = jnp.zeros((_BLK, 1), jnp.uint32)
    for bit in range(31, -1, -1):
        cand = prefix | jnp.uint32(1 << bit)
        cnt = jnp.sum((ukey >= cand).astype(jnp.int32), axis=1, keepdims=True)
        prefix = jnp.where(cnt >= _KEEP, cand, prefix)
    keep = ukey >= prefix

    m = jnp.max(scores, axis=1, keepdims=True)
    p = jnp.where(keep, jnp.exp(scores - m), 0.0)
    attn = p / jnp.sum(p, axis=1, keepdims=True)
    o_ref[0] = jnp.dot(attn, v_s[...], preferred_element_type=jnp.float32)


def _attention(x2, wq, bq3, wk, bk3, wv, bv3):
    w_spec = pl.BlockSpec((1, _DH, _DH), lambda h, i: (h, 0, 0))
    b_spec = pl.BlockSpec((1, 1, _DH), lambda h, i: (h, 0, 0))
    return pl.pallas_call(
        _attn_kernel,
        grid=(_H, _NB),
        in_specs=[
            pl.BlockSpec((_S, _DH), lambda h, i: (0, h)),
            w_spec, b_spec, w_spec, b_spec, w_spec, b_spec,
        ],
        out_specs=pl.BlockSpec((1, _BLK, _DH), lambda h, i: (h, i, 0)),
        out_shape=jax.ShapeDtypeStruct((_H, _S, _DH), jnp.float32),
        scratch_shapes=[
            pltpu.VMEM((_S, _DH), jnp.float32),
            pltpu.VMEM((_S, _DH), jnp.float32),
        ],
    )(x2, wq, bq3, wk, bk3, wv, bv3)


_TM = 256
_TN = 256


def _proj_kernel(a_ref, wo_ref, bo_ref, o_ref):
    o_ref[...] = jax.lax.dot_general(
        a_ref[...], wo_ref[...], (((1,), (1,)), ((), ())),
        preferred_element_type=jnp.float32) + bo_ref[0]


def _proj(a, wo, bo2):
    return pl.pallas_call(
        _proj_kernel,
        grid=(_DIM // _TN, _S // _TM),
        in_specs=[
            pl.BlockSpec((_TM, _DIM), lambda tj, ti: (ti, 0)),
            pl.BlockSpec((_TN, _DIM), lambda tj, ti: (tj, 0)),
            pl.BlockSpec((1, _TN), lambda tj, ti: (0, tj)),
        ],
        out_specs=pl.BlockSpec((_TM, _TN), lambda tj, ti: (ti, tj)),
        out_shape=jax.ShapeDtypeStruct((_S, _DIM), jnp.float32),
    )(a, wo, bo2)


def kernel(x, causal_mask, Wq, bq, Wk, bk, Wv, bv, Wo, bo):
    x2 = x.reshape(_S, _DIM)
    out = _attention(
        x2, Wq, bq.reshape(_H, 1, _DH),
        Wk, bk.reshape(_H, 1, _DH),
        Wv, bv.reshape(_H, 1, _DH))
    a = out.transpose(0, 2, 1).reshape(_DIM, _S)
    final = _proj(a, Wo, bo.reshape(1, _DIM))
    return final.reshape(1, _S, _DIM)
